# Initial kernel scaffold; baseline (speedup 1.0000x reference)
import jax
import jax.numpy as jnp
from jax.experimental import pallas as pl


def kernel(state, root_W1, root_b1, root_W2, root_b2, exp_W1, exp_b1, exp_W2, exp_b2):
    raise NotImplementedError("write your pallas kernel here")


# speedup vs baseline: 2.3843x; 2.3843x over previous
import functools

import jax
import jax.numpy as jnp
from jax.experimental import pallas as pl
from jax.experimental.pallas import tpu as pltpu

E = 16
L = 1024
D = 1024
H = 512
K = 10
B = 1024

TB = 128
NT = B // TB
NITEMS = NT + E - 1
NEG = -jnp.inf

def _dot(a, b):
    return jnp.dot(a, b, preferred_element_type=jnp.float32)


def _dotx(a, b):
    return jnp.dot(a, b, preferred_element_type=jnp.float32,
                   precision=jax.lax.Precision.HIGHEST)


def _router_body(x_ref, w1_ref, b1_ref, w2_ref, b2_ref,
                 sx_ref, idx_ref, idxs_ref, rls_ref, pos_ref,
                 ie_ref, it_ref, ilo_ref, ihi_ref):
    x = x_ref[...]
    h = jnp.maximum(_dot(x, w1_ref[...]) + b1_ref[...], 0.0)
    logits = _dot(h, w2_ref[...])[:, :E] + b2_ref[...]

    m = jnp.max(logits, axis=1, keepdims=True)
    lse = m + jnp.log(jnp.sum(jnp.exp(logits - m), axis=1, keepdims=True))
    ce = jax.lax.broadcasted_iota(jnp.int32, (B, E), 1)
    idx = jnp.min(jnp.where(logits == m, ce, E), axis=1, keepdims=True)
    idx_ref[...] = idx
    rls = m - lse

    oh = (ce == idx).astype(jnp.float32)
    counts = jnp.sum(oh, axis=0, keepdims=True)
    ree = jax.lax.broadcasted_iota(jnp.int32, (E, E), 0)
    cee = jax.lax.broadcasted_iota(jnp.int32, (E, E), 1)
    starts = _dotx(counts, (ree < cee).astype(jnp.float32))
    ends = starts + counts

    rr = jax.lax.broadcasted_iota(jnp.int32, (B, B), 0)
    cc = jax.lax.broadcasted_iota(jnp.int32, (B, B), 1)
    ltb = (cc < rr).astype(jnp.float32)
    ranks = _dotx(ltb, oh)
    rank = jnp.sum(ranks * oh, axis=1, keepdims=True)
    start_i = jnp.sum(starts * oh, axis=1, keepdims=True)
    posf = start_i + rank
    pos_ref[...] = posf.astype(jnp.int32)

    diag = jnp.where(rr == cc, posf, 0.0)
    pos_row = _dotx(jnp.ones((1, B), jnp.float32), diag)
    perm = (pos_row.astype(jnp.int32) == rr).astype(jnp.float32)
    sx_ref[...] = _dotx(perm, x)
    idxs_ref[...] = jnp.round(_dotx(perm, idx.astype(jnp.float32))).astype(jnp.int32)
    rls_ref[...] = _dotx(perm, rls)

    nf = E * NT
    fc = jax.lax.broadcasted_iota(jnp.int32, (nf, 1), 0)
    ec = fc // NT
    tc = fc - ec * NT
    ohe = (jax.lax.broadcasted_iota(jnp.int32, (nf, E), 1) == ec).astype(jnp.float32)
    st_c = jnp.sum(ohe * starts, axis=1, keepdims=True)
    en_c = jnp.sum(ohe * ends, axis=1, keepdims=True)
    lo_c = jnp.maximum(st_c, (tc * TB).astype(jnp.float32))
    hi_c = jnp.minimum(en_c, ((tc + 1) * TB).astype(jnp.float32))
    act_c = (lo_c < hi_c).astype(jnp.float32)

    rf = jax.lax.broadcasted_iota(jnp.int32, (nf, nf), 0)
    cf = jax.lax.broadcasted_iota(jnp.int32, (nf, nf), 1)
    diag_a = jnp.where(rf == cf, act_c, 0.0)
    act_row = _dotx(jnp.ones((1, nf), jnp.float32), diag_a)
    cix_row = _dotx(act_row, (rf < cf).astype(jnp.float32))
    total = jnp.sum(act_c)

    selt = ((cix_row == rf.astype(jnp.float32)) & (act_row > 0.5)).astype(jnp.float32)
    vals = jnp.concatenate([ec.astype(jnp.float32), tc.astype(jnp.float32),
                            lo_c, hi_c], axis=1)
    items = _dotx(selt, vals)
    sc = jax.lax.broadcasted_iota(jnp.int32, (nf, 1), 0).astype(jnp.float32)
    last = jnp.sum(jnp.where(sc == total - 1.0, items, 0.0), axis=0, keepdims=True)
    pad = sc >= total
    ie_ref[...] = jnp.round(jnp.where(pad, last[0, 0], items[:, 0:1])).astype(jnp.int32)
    it_ref[...] = jnp.round(jnp.where(pad, last[0, 1], items[:, 1:2])).astype(jnp.int32)
    ilo_ref[...] = jnp.round(jnp.where(pad, 0.0, items[:, 2:3])).astype(jnp.int32)
    ihi_ref[...] = jnp.round(jnp.where(pad, 0.0, items[:, 3:4])).astype(jnp.int32)


def _expert_body(ie_ref, it_ref, ilo_ref, ihi_ref,
                 sx_ref, w1_ref, b1_ref, w2_ref, b2_ref, out_ref):
    i = pl.program_id(0)
    t = it_ref[i]
    lo = ilo_ref[i]
    hi = ihi_ref[i]
    x = sx_ref[pl.ds(t * TB, TB), :]
    h = jnp.maximum(_dot(x, w1_ref[0]) + b1_ref[0], 0.0)
    le = _dot(h, w2_ref[0]) + b2_ref[0]
    g = t * TB + jax.lax.broadcasted_iota(jnp.int32, (TB, 1), 0)
    mask = (g >= lo) & (g < hi)
    cur = out_ref[pl.ds(t * TB, TB), :]
    out_ref[pl.ds(t * TB, TB), :] = jnp.where(mask, le, cur)


def _finish_body(fs_ref, idxs_ref, rls_ref, pos_ref, out_ref):
    l0 = fs_ref[...]
    m0 = jnp.max(l0, axis=1, keepdims=True)
    lse = m0 + jnp.log(jnp.sum(jnp.exp(l0 - m0), axis=1, keepdims=True))
    cols = jax.lax.broadcasted_iota(jnp.int32, (B, L), 1)

    lcur = l0
    vals, cands = [], []
    for _ in range(K + 1):
        mv = jnp.max(lcur, axis=1, keepdims=True)
        am = jnp.min(jnp.where(lcur == mv, cols, L), axis=1, keepdims=True)
        vals.append(mv)
        cands.append(am)
        lcur = jnp.where(cols == am, NEG, lcur)
    v11 = jnp.concatenate(vals, axis=1)
    c11 = jnp.concatenate(cands, axis=1)

    inval = (idxs_ref[...] == 0) & (c11 == 0)
    j11 = jax.lax.broadcasted_iota(jnp.int32, (B, K + 1), 1)
    pinv = jnp.min(jnp.where(inval, j11, K + 1), axis=1, keepdims=True)
    j10 = jax.lax.broadcasted_iota(jnp.int32, (B, K), 1)
    shift = j10 >= pinv
    kept = jnp.where(shift, c11[:, 1:K + 1], c11[:, :K])
    keptv = jnp.where(shift, v11[:, 1:K + 1], v11[:, :K])
    olp = rls_ref[...] + (keptv - lse)

    ccb = jax.lax.broadcasted_iota(jnp.int32, (B, B), 1)
    u = (ccb == pos_ref[...]).astype(jnp.float32)
    payload = jnp.concatenate(
        [olp, kept.astype(jnp.float32),
         jnp.zeros((B, 12), jnp.float32)], axis=1)
    out_ref[...] = _dotx(u, payload)


@jax.jit
def kernel(state, root_W1, root_b1, root_W2, root_b2,
           exp_W1, exp_b1, exp_W2, exp_b2):
    f32 = jnp.float32
    i32 = jnp.int32
    nf = E * NT

    router = pl.pallas_call(
        _router_body,
        out_shape=(
            jax.ShapeDtypeStruct((B, D), f32),
            jax.ShapeDtypeStruct((B, 1), i32),
            jax.ShapeDtypeStruct((B, 1), i32),
            jax.ShapeDtypeStruct((B, 1), f32),
            jax.ShapeDtypeStruct((B, 1), i32),
            jax.ShapeDtypeStruct((nf, 1), i32),
            jax.ShapeDtypeStruct((nf, 1), i32),
            jax.ShapeDtypeStruct((nf, 1), i32),
            jax.ShapeDtypeStruct((nf, 1), i32),
        ),
    )
    root_W2p = jnp.concatenate(
        [root_W2, jnp.zeros((H, 128 - E), f32)], axis=1)
    sx, idx, idxs, rls, pos, ie, it, ilo, ihi = router(
        state, root_W1, root_b1.reshape(1, H), root_W2p, root_b2.reshape(1, E))

    ie1 = ie[:NITEMS, 0]
    it1 = it[:NITEMS, 0]
    ilo1 = ilo[:NITEMS, 0]
    ihi1 = ihi[:NITEMS, 0]

    grouped = pl.pallas_call(
        _expert_body,
        grid_spec=pltpu.PrefetchScalarGridSpec(
            num_scalar_prefetch=4,
            grid=(NITEMS,),
            in_specs=[
                pl.BlockSpec((B, D), lambda i, *_: (0, 0)),
                pl.BlockSpec((1, D, H), lambda i, ie, it, lo, hi: (ie[i], 0, 0)),
                pl.BlockSpec((1, 1, H), lambda i, ie, it, lo, hi: (ie[i], 0, 0)),
                pl.BlockSpec((1, H, L), lambda i, ie, it, lo, hi: (ie[i], 0, 0)),
                pl.BlockSpec((1, 1, L), lambda i, ie, it, lo, hi: (ie[i], 0, 0)),
            ],
            out_specs=pl.BlockSpec((B, L), lambda i, *_: (0, 0)),
        ),
        out_shape=jax.ShapeDtypeStruct((B, L), f32),
    )
    fs = grouped(ie1, it1, ilo1, ihi1, sx,
                 exp_W1, exp_b1.reshape(E, 1, H),
                 exp_W2, exp_b2.reshape(E, 1, L))

    finish = pl.pallas_call(
        _finish_body,
        out_shape=jax.ShapeDtypeStruct((B, 32), f32),
    )
    payload = finish(fs, idxs, rls, pos)

    out_lp = payload[:, :K]
    leaf = jnp.round(payload[:, K:2 * K]).astype(i32)
    branch = jnp.broadcast_to(idx, (B, K))
    trajectories = jnp.stack([branch, leaf], axis=-1)
    return trajectories, out_lp


# speedup vs baseline: 2.5067x; 1.0513x over previous
import functools

import jax
import jax.numpy as jnp
from jax.experimental import pallas as pl
from jax.experimental.pallas import tpu as pltpu
from jax.experimental.pallas import tpu_sc as plsc

_SC_WORKERS = 32


def _sc_gather(table, ids):
    bn, dn = table.shape
    bpw = bn // _SC_WORKERS
    mesh = plsc.VectorSubcoreMesh(core_axis_name="c", subcore_axis_name="s")

    @functools.partial(
        pl.kernel, mesh=mesh,
        out_type=jax.ShapeDtypeStruct((bn, dn), table.dtype),
        scratch_types=[
            pltpu.VMEM((bpw,), jnp.int32),
            pltpu.VMEM((bpw, dn), table.dtype),
            pltpu.SemaphoreType.DMA,
        ],
    )
    def k(table_hbm, ids_hbm, out_hbm, ids_v, rows_v, sem):
        wid = jax.lax.axis_index("s") * 2 + jax.lax.axis_index("c")
        base = wid * bpw
        pltpu.sync_copy(ids_hbm.at[pl.ds(base, bpw)], ids_v)
        pltpu.async_copy(table_hbm.at[ids_v], rows_v, sem).wait()
        pltpu.sync_copy(rows_v, out_hbm.at[pl.ds(base, bpw)])

    return k(table, ids)

E = 16
L = 1024
D = 1024
H = 512
K = 10
B = 1024

TB = 128
NT = B // TB
NITEMS = NT + E - 1
NEG = -jnp.inf

def _dot(a, b):
    return jnp.dot(a, b, preferred_element_type=jnp.float32)


def _dotx(a, b):
    return jnp.dot(a, b, preferred_element_type=jnp.float32,
                   precision=jax.lax.Precision.HIGHEST)


def _router_body(x_ref, w1_ref, b1_ref, w2_ref, b2_ref,
                 sid_ref, idx_ref, idxs_ref, rls_ref, pos_ref,
                 ie_ref, it_ref, ilo_ref, ihi_ref):
    x = x_ref[...]
    h = jnp.maximum(_dot(x, w1_ref[...]) + b1_ref[...], 0.0)
    logits = _dot(h, w2_ref[...])[:, :E] + b2_ref[...]

    m = jnp.max(logits, axis=1, keepdims=True)
    lse = m + jnp.log(jnp.sum(jnp.exp(logits - m), axis=1, keepdims=True))
    ce = jax.lax.broadcasted_iota(jnp.int32, (B, E), 1)
    idx = jnp.min(jnp.where(logits == m, ce, E), axis=1, keepdims=True)
    idx_ref[...] = idx
    rls = m - lse

    oh = (ce == idx).astype(jnp.float32)
    counts = jnp.sum(oh, axis=0, keepdims=True)
    ree = jax.lax.broadcasted_iota(jnp.int32, (E, E), 0)
    cee = jax.lax.broadcasted_iota(jnp.int32, (E, E), 1)
    starts = _dotx(counts, (ree < cee).astype(jnp.float32))
    ends = starts + counts

    rr = jax.lax.broadcasted_iota(jnp.int32, (B, B), 0)
    cc = jax.lax.broadcasted_iota(jnp.int32, (B, B), 1)
    ltb = (cc < rr).astype(jnp.float32)
    ranks = _dotx(ltb, oh)
    rank = jnp.sum(ranks * oh, axis=1, keepdims=True)
    start_i = jnp.sum(starts * oh, axis=1, keepdims=True)
    posf = start_i + rank
    pos_ref[...] = posf.astype(jnp.int32)

    diag = jnp.where(rr == cc, posf, 0.0)
    pos_row = _dotx(jnp.ones((1, B), jnp.float32), diag)
    perm = (pos_row.astype(jnp.int32) == rr).astype(jnp.float32)
    rowids = jax.lax.broadcasted_iota(jnp.int32, (B, 1), 0).astype(jnp.float32)
    srt = _dotx(perm, jnp.concatenate(
        [rowids, idx.astype(jnp.float32), rls], axis=1))
    sid_ref[...] = jnp.round(srt[:, 0:1]).astype(jnp.int32)
    idxs_ref[...] = jnp.round(srt[:, 1:2]).astype(jnp.int32)
    rls_ref[...] = srt[:, 2:3]

    nf = E * NT
    fc = jax.lax.broadcasted_iota(jnp.int32, (nf, 1), 0)
    ec = fc // NT
    tc = fc - ec * NT
    ohe = (jax.lax.broadcasted_iota(jnp.int32, (nf, E), 1) == ec).astype(jnp.float32)
    st_c = jnp.sum(ohe * starts, axis=1, keepdims=True)
    en_c = jnp.sum(ohe * ends, axis=1, keepdims=True)
    lo_c = jnp.maximum(st_c, (tc * TB).astype(jnp.float32))
    hi_c = jnp.minimum(en_c, ((tc + 1) * TB).astype(jnp.float32))
    act_c = (lo_c < hi_c).astype(jnp.float32)

    rf = jax.lax.broadcasted_iota(jnp.int32, (nf, nf), 0)
    cf = jax.lax.broadcasted_iota(jnp.int32, (nf, nf), 1)
    diag_a = jnp.where(rf == cf, act_c, 0.0)
    act_row = _dotx(jnp.ones((1, nf), jnp.float32), diag_a)
    cix_row = _dotx(act_row, (rf < cf).astype(jnp.float32))
    total = jnp.sum(act_c)

    selt = ((cix_row == rf.astype(jnp.float32)) & (act_row > 0.5)).astype(jnp.float32)
    vals = jnp.concatenate([ec.astype(jnp.float32), tc.astype(jnp.float32),
                            lo_c, hi_c], axis=1)
    items = _dotx(selt, vals)
    sc = jax.lax.broadcasted_iota(jnp.int32, (nf, 1), 0).astype(jnp.float32)
    last = jnp.sum(jnp.where(sc == total - 1.0, items, 0.0), axis=0, keepdims=True)
    pad = sc >= total
    ie_ref[...] = jnp.round(jnp.where(pad, last[0, 0], items[:, 0:1])).astype(jnp.int32)
    it_ref[...] = jnp.round(jnp.where(pad, last[0, 1], items[:, 1:2])).astype(jnp.int32)
    ilo_ref[...] = jnp.round(jnp.where(pad, 0.0, items[:, 2:3])).astype(jnp.int32)
    ihi_ref[...] = jnp.round(jnp.where(pad, 0.0, items[:, 3:4])).astype(jnp.int32)


def _expert_body(ie_ref, it_ref, ilo_ref, ihi_ref,
                 sx_ref, w1_ref, b1_ref, w2_ref, b2_ref, out_ref):
    i = pl.program_id(0)
    t = it_ref[i]
    lo = ilo_ref[i]
    hi = ihi_ref[i]
    x = sx_ref[pl.ds(t * TB, TB), :]
    h = jnp.maximum(_dot(x, w1_ref[0]) + b1_ref[0], 0.0)
    le = _dot(h, w2_ref[0]) + b2_ref[0]
    g = t * TB + jax.lax.broadcasted_iota(jnp.int32, (TB, 1), 0)
    mask = (g >= lo) & (g < hi)
    cur = out_ref[pl.ds(t * TB, TB), :]
    out_ref[pl.ds(t * TB, TB), :] = jnp.where(mask, le, cur)


def _finish_body(fs_ref, idxs_ref, rls_ref, out_ref):
    l0 = fs_ref[...]
    m0 = jnp.max(l0, axis=1, keepdims=True)
    lse = m0 + jnp.log(jnp.sum(jnp.exp(l0 - m0), axis=1, keepdims=True))
    cols = jax.lax.broadcasted_iota(jnp.int32, (B, L), 1)

    lcur = l0
    vals, cands = [], []
    for _ in range(K + 1):
        mv = jnp.max(lcur, axis=1, keepdims=True)
        am = jnp.min(jnp.where(lcur == mv, cols, L), axis=1, keepdims=True)
        vals.append(mv)
        cands.append(am)
        lcur = jnp.where(cols == am, NEG, lcur)
    v11 = jnp.concatenate(vals, axis=1)
    c11 = jnp.concatenate(cands, axis=1)

    inval = (idxs_ref[...] == 0) & (c11 == 0)
    j11 = jax.lax.broadcasted_iota(jnp.int32, (B, K + 1), 1)
    pinv = jnp.min(jnp.where(inval, j11, K + 1), axis=1, keepdims=True)
    j10 = jax.lax.broadcasted_iota(jnp.int32, (B, K), 1)
    shift = j10 >= pinv
    kept = jnp.where(shift, c11[:, 1:K + 1], c11[:, :K])
    keptv = jnp.where(shift, v11[:, 1:K + 1], v11[:, :K])
    olp = rls_ref[...] + (keptv - lse)

    out_ref[...] = jnp.concatenate(
        [olp, kept.astype(jnp.float32),
         jnp.zeros((B, 108), jnp.float32)], axis=1)


@jax.jit
def kernel(state, root_W1, root_b1, root_W2, root_b2,
           exp_W1, exp_b1, exp_W2, exp_b2):
    f32 = jnp.float32
    i32 = jnp.int32
    nf = E * NT

    router = pl.pallas_call(
        _router_body,
        out_shape=(
            jax.ShapeDtypeStruct((B, 1), i32),
            jax.ShapeDtypeStruct((B, 1), i32),
            jax.ShapeDtypeStruct((B, 1), i32),
            jax.ShapeDtypeStruct((B, 1), f32),
            jax.ShapeDtypeStruct((B, 1), i32),
            jax.ShapeDtypeStruct((nf, 1), i32),
            jax.ShapeDtypeStruct((nf, 1), i32),
            jax.ShapeDtypeStruct((nf, 1), i32),
            jax.ShapeDtypeStruct((nf, 1), i32),
        ),
    )
    root_W2p = jnp.concatenate(
        [root_W2, jnp.zeros((H, 128 - E), f32)], axis=1)
    sid, idx, idxs, rls, pos, ie, it, ilo, ihi = router(
        state, root_W1, root_b1.reshape(1, H), root_W2p, root_b2.reshape(1, E))
    sx = _sc_gather(state, sid.reshape(B))

    ie1 = ie[:NITEMS, 0]
    it1 = it[:NITEMS, 0]
    ilo1 = ilo[:NITEMS, 0]
    ihi1 = ihi[:NITEMS, 0]

    grouped = pl.pallas_call(
        _expert_body,
        grid_spec=pltpu.PrefetchScalarGridSpec(
            num_scalar_prefetch=4,
            grid=(NITEMS,),
            in_specs=[
                pl.BlockSpec((B, D), lambda i, *_: (0, 0)),
                pl.BlockSpec((1, D, H), lambda i, ie, it, lo, hi: (ie[i], 0, 0)),
                pl.BlockSpec((1, 1, H), lambda i, ie, it, lo, hi: (ie[i], 0, 0)),
                pl.BlockSpec((1, H, L), lambda i, ie, it, lo, hi: (ie[i], 0, 0)),
                pl.BlockSpec((1, 1, L), lambda i, ie, it, lo, hi: (ie[i], 0, 0)),
            ],
            out_specs=pl.BlockSpec((B, L), lambda i, *_: (0, 0)),
        ),
        out_shape=jax.ShapeDtypeStruct((B, L), f32),
    )
    fs = grouped(ie1, it1, ilo1, ihi1, sx,
                 exp_W1, exp_b1.reshape(E, 1, H),
                 exp_W2, exp_b2.reshape(E, 1, L))

    finish = pl.pallas_call(
        _finish_body,
        out_shape=jax.ShapeDtypeStruct((B, 128), f32),
    )
    payload_sorted = finish(fs, idxs, rls)
    payload = _sc_gather(payload_sorted, pos.reshape(B))

    out_lp = payload[:, :K]
    leaf = jnp.round(payload[:, K:2 * K]).astype(i32)
    branch = jnp.broadcast_to(idx, (B, K))
    trajectories = jnp.stack([branch, leaf], axis=-1)
    return trajectories, out_lp


# speedup vs baseline: 2.9318x; 1.1696x over previous
import functools

import jax
import jax.numpy as jnp
from jax.experimental import pallas as pl
from jax.experimental.pallas import tpu as pltpu
from jax.experimental.pallas import tpu_sc as plsc

E = 16
L = 1024
D = 1024
H = 512
K = 10
B = 1024

TB = 128
NT = B // TB
NITEMS = NT + E - 1
NEG = -jnp.inf

_SC_WORKERS = 32


def _dot(a, b):
    return jnp.dot(a, b, preferred_element_type=jnp.float32)


def _dotx(a, b):
    return jnp.dot(a, b, preferred_element_type=jnp.float32,
                   precision=jax.lax.Precision.HIGHEST)


def _sc_gather(table, ids):
    bn, dn = table.shape
    bpw = bn // _SC_WORKERS
    mesh = plsc.VectorSubcoreMesh(core_axis_name="c", subcore_axis_name="s")

    @functools.partial(
        pl.kernel, mesh=mesh,
        out_type=jax.ShapeDtypeStruct((bn, dn), table.dtype),
        scratch_types=[
            pltpu.VMEM((bpw,), jnp.int32),
            pltpu.VMEM((bpw, dn), table.dtype),
            pltpu.SemaphoreType.DMA,
        ],
    )
    def k(table_hbm, ids_hbm, out_hbm, ids_v, rows_v, sem):
        wid = jax.lax.axis_index("s") * 2 + jax.lax.axis_index("c")
        base = wid * bpw
        pltpu.sync_copy(ids_hbm.at[pl.ds(base, bpw)], ids_v)
        pltpu.async_copy(table_hbm.at[ids_v], rows_v, sem).wait()
        pltpu.sync_copy(rows_v, out_hbm.at[pl.ds(base, bpw)])

    return k(table, ids)


def _row_to_lane(colvec, rr, cc):
    n = colvec.shape[0]
    diag = jnp.where(rr == cc, colvec, 0.0)
    return _dotx(jnp.ones((1, n), jnp.float32), diag)


def _router_body(x_ref, w1_ref, b1_ref, w2_ref, b2_ref,
                 sid_ref, idx_ref, rls_ref, pos_ref, end0_ref,
                 ie_ref, it_ref, ilo_ref, ihi_ref):
    x = x_ref[...]
    h = jnp.maximum(_dot(x, w1_ref[...]) + b1_ref[...], 0.0)
    logits = _dot(h, w2_ref[...])[:, :E] + b2_ref[...]

    m = jnp.max(logits, axis=1, keepdims=True)
    lse = m + jnp.log(jnp.sum(jnp.exp(logits - m), axis=1, keepdims=True))
    ce = jax.lax.broadcasted_iota(jnp.int32, (B, E), 1)
    idx = jnp.min(jnp.where(logits == m, ce, E), axis=1, keepdims=True)
    idx_ref[...] = idx
    rls = m - lse

    oh = (ce == idx).astype(jnp.float32)
    counts = jnp.sum(oh, axis=0, keepdims=True)
    ree = jax.lax.broadcasted_iota(jnp.int32, (E, E), 0)
    cee = jax.lax.broadcasted_iota(jnp.int32, (E, E), 1)
    starts = _dotx(counts, (ree < cee).astype(jnp.float32))
    ends = starts + counts
    end0_ref[...] = counts[0:1, 0:1].astype(jnp.int32)

    rr = jax.lax.broadcasted_iota(jnp.int32, (B, B), 0)
    cc = jax.lax.broadcasted_iota(jnp.int32, (B, B), 1)
    idxf = idx.astype(jnp.float32)
    idx_row = _row_to_lane(idxf, rr, cc)
    eq = (idx_row == idxf) & (cc < rr)
    rank = jnp.sum(eq.astype(jnp.float32), axis=1, keepdims=True)
    start_i = jnp.sum(starts * oh, axis=1, keepdims=True)
    posf = start_i + rank
    pos_ref[...] = posf.astype(jnp.int32)

    pos_row = _row_to_lane(posf, rr, cc)
    perm = (pos_row.astype(jnp.int32) == rr).astype(jnp.float32)
    sid_ref[...] = jnp.round(
        jnp.sum(perm * cc.astype(jnp.float32), axis=1, keepdims=True)).astype(jnp.int32)
    rls_row = _row_to_lane(rls, rr, cc)
    rls_ref[...] = jnp.sum(perm * rls_row, axis=1, keepdims=True)

    nf = E * NT
    fc = jax.lax.broadcasted_iota(jnp.int32, (nf, 1), 0)
    ec = fc // NT
    tc = fc - ec * NT
    ohe = (jax.lax.broadcasted_iota(jnp.int32, (nf, E), 1) == ec).astype(jnp.float32)
    st_c = jnp.sum(ohe * starts, axis=1, keepdims=True)
    en_c = jnp.sum(ohe * ends, axis=1, keepdims=True)
    lo_c = jnp.maximum(st_c, (tc * TB).astype(jnp.float32))
    hi_c = jnp.minimum(en_c, ((tc + 1) * TB).astype(jnp.float32))
    act_c = (lo_c < hi_c).astype(jnp.float32)

    rf = jax.lax.broadcasted_iota(jnp.int32, (nf, nf), 0)
    cf = jax.lax.broadcasted_iota(jnp.int32, (nf, nf), 1)
    act_row = _row_to_lane(act_c, rf, cf)
    cix_row = _dotx(act_row, (rf < cf).astype(jnp.float32))
    total = jnp.sum(act_c)

    selt = ((cix_row == rf.astype(jnp.float32)) & (act_row > 0.5)).astype(jnp.float32)
    vals = jnp.concatenate([ec.astype(jnp.float32), tc.astype(jnp.float32),
                            lo_c, hi_c], axis=1)
    items = _dotx(selt, vals)
    sc = jax.lax.broadcasted_iota(jnp.int32, (nf, 1), 0).astype(jnp.float32)
    last = jnp.sum(jnp.where(sc == total - 1.0, items, 0.0), axis=0, keepdims=True)
    pad = sc >= total
    ie_ref[...] = jnp.round(jnp.where(pad, last[0, 0], items[:, 0:1])).astype(jnp.int32)
    it_ref[...] = jnp.round(jnp.where(pad, last[0, 1], items[:, 1:2])).astype(jnp.int32)
    ilo_ref[...] = jnp.round(jnp.where(pad, 0.0, items[:, 2:3])).astype(jnp.int32)
    ihi_ref[...] = jnp.round(jnp.where(pad, 0.0, items[:, 3:4])).astype(jnp.int32)


def _expert_finish_body(ie_ref, it_ref, ilo_ref, ihi_ref, e0_ref,
                        sx_ref, w1_ref, b1_ref, w2_ref, b2_ref,
                        rls_ref, pos_ref, out_ref, fs_ref):
    i = pl.program_id(0)
    t = it_ref[i]
    lo = ilo_ref[i]
    hi = ihi_ref[i]

    @pl.when(lo < hi)
    def _compute():
        x = sx_ref[pl.ds(t * TB, TB), :]
        h = jnp.maximum(_dot(x, w1_ref[0]) + b1_ref[0], 0.0)
        le = _dot(h, w2_ref[0]) + b2_ref[0]
        g = t * TB + jax.lax.broadcasted_iota(jnp.int32, (TB, 1), 0)
        mask = (g >= lo) & (g < hi)
        cur = fs_ref[pl.ds(t * TB, TB), :]
        fs_ref[pl.ds(t * TB, TB), :] = jnp.where(mask, le, cur)

    @pl.when(i == NITEMS - 1)
    def _finish():
        l0 = fs_ref[...]
        m0 = jnp.max(l0, axis=1, keepdims=True)
        lse = m0 + jnp.log(jnp.sum(jnp.exp(l0 - m0), axis=1, keepdims=True))
        cols = jax.lax.broadcasted_iota(jnp.int32, (B, L), 1)

        lcur = l0
        vals, cands = [], []
        for _ in range(K + 1):
            mv = jnp.max(lcur, axis=1, keepdims=True)
            am = jnp.min(jnp.where(lcur == mv, cols, L), axis=1, keepdims=True)
            vals.append(mv)
            cands.append(am)
            lcur = jnp.where(cols == am, NEG, lcur)
        v11 = jnp.concatenate(vals, axis=1)
        c11 = jnp.concatenate(cands, axis=1)

        rowi = jax.lax.broadcasted_iota(jnp.int32, (B, 1), 0)
        inval = (rowi < e0_ref[0]) & (c11 == 0)
        j11 = jax.lax.broadcasted_iota(jnp.int32, (B, K + 1), 1)
        pinv = jnp.min(jnp.where(inval, j11, K + 1), axis=1, keepdims=True)
        j10 = jax.lax.broadcasted_iota(jnp.int32, (B, K), 1)
        shift = j10 >= pinv
        kept = jnp.where(shift, c11[:, 1:K + 1], c11[:, :K])
        keptv = jnp.where(shift, v11[:, 1:K + 1], v11[:, :K])
        olp = rls_ref[...] + (keptv - lse)

        ccb = jax.lax.broadcasted_iota(jnp.int32, (B, B), 1)
        u = (ccb == pos_ref[...]).astype(jnp.float32)
        payload = jnp.concatenate(
            [olp, kept.astype(jnp.float32),
             jnp.zeros((B, 12), jnp.float32)], axis=1)
        out_ref[...] = _dotx(u, payload)


@jax.jit
def kernel(state, root_W1, root_b1, root_W2, root_b2,
           exp_W1, exp_b1, exp_W2, exp_b2):
    f32 = jnp.float32
    i32 = jnp.int32
    nf = E * NT

    router = pl.pallas_call(
        _router_body,
        out_shape=(
            jax.ShapeDtypeStruct((B, 1), i32),
            jax.ShapeDtypeStruct((B, 1), i32),
            jax.ShapeDtypeStruct((B, 1), f32),
            jax.ShapeDtypeStruct((B, 1), i32),
            jax.ShapeDtypeStruct((1, 1), i32),
            jax.ShapeDtypeStruct((nf, 1), i32),
            jax.ShapeDtypeStruct((nf, 1), i32),
            jax.ShapeDtypeStruct((nf, 1), i32),
            jax.ShapeDtypeStruct((nf, 1), i32),
        ),
    )
    root_W2p = jnp.concatenate(
        [root_W2, jnp.zeros((H, 128 - E), f32)], axis=1)
    sid, idx, rls, pos, end0, ie, it, ilo, ihi = router(
        state, root_W1, root_b1.reshape(1, H), root_W2p, root_b2.reshape(1, E))
    sx = _sc_gather(state, sid.reshape(B))

    fused = pl.pallas_call(
        _expert_finish_body,
        grid_spec=pltpu.PrefetchScalarGridSpec(
            num_scalar_prefetch=5,
            grid=(NITEMS,),
            in_specs=[
                pl.BlockSpec((B, D), lambda i, *_: (0, 0)),
                pl.BlockSpec((1, D, H), lambda i, ie, it, lo, hi, e0: (ie[i], 0, 0)),
                pl.BlockSpec((1, 1, H), lambda i, ie, it, lo, hi, e0: (ie[i], 0, 0)),
                pl.BlockSpec((1, H, L), lambda i, ie, it, lo, hi, e0: (ie[i], 0, 0)),
                pl.BlockSpec((1, 1, L), lambda i, ie, it, lo, hi, e0: (ie[i], 0, 0)),
                pl.BlockSpec((B, 1), lambda i, *_: (0, 0)),
                pl.BlockSpec((B, 1), lambda i, *_: (0, 0)),
            ],
            out_specs=pl.BlockSpec((B, 32), lambda i, *_: (0, 0)),
            scratch_shapes=[pltpu.VMEM((B, L), f32)],
        ),
        out_shape=jax.ShapeDtypeStruct((B, 32), f32),
    )
    payload = fused(ie[:NITEMS, 0], it[:NITEMS, 0], ilo[:NITEMS, 0],
                    ihi[:NITEMS, 0], end0.reshape(1), sx,
                    exp_W1, exp_b1.reshape(E, 1, H),
                    exp_W2, exp_b2.reshape(E, 1, L),
                    rls, pos)

    out_lp = payload[:, :K]
    leaf = jnp.round(payload[:, K:2 * K]).astype(i32)
    branch = jnp.broadcast_to(idx, (B, K))
    trajectories = jnp.stack([branch, leaf], axis=-1)
    return trajectories, out_lp


# speedup vs baseline: 3.0106x; 1.0269x over previous
import functools

import jax
import jax.numpy as jnp
from jax.experimental import pallas as pl
from jax.experimental.pallas import tpu as pltpu
from jax.experimental.pallas import tpu_sc as plsc

E = 16
L = 1024
D = 1024
H = 512
K = 10
B = 1024

TB = 256
NT = B // TB
NITEMS = NT + E - 1
NEG = -jnp.inf

_SC_WORKERS = 32


def _dot(a, b):
    return jnp.dot(a, b, preferred_element_type=jnp.float32)


def _dotx(a, b):
    return jnp.dot(a, b, preferred_element_type=jnp.float32,
                   precision=jax.lax.Precision.HIGHEST)


def _sc_gather(table, ids):
    bn, dn = table.shape
    bpw = bn // _SC_WORKERS
    mesh = plsc.VectorSubcoreMesh(core_axis_name="c", subcore_axis_name="s")

    @functools.partial(
        pl.kernel, mesh=mesh,
        out_type=jax.ShapeDtypeStruct((bn, dn), table.dtype),
        scratch_types=[
            pltpu.VMEM((bpw,), jnp.int32),
            pltpu.VMEM((bpw, dn), table.dtype),
            pltpu.SemaphoreType.DMA,
        ],
    )
    def k(table_hbm, ids_hbm, out_hbm, ids_v, rows_v, sem):
        wid = jax.lax.axis_index("s") * 2 + jax.lax.axis_index("c")
        base = wid * bpw
        pltpu.sync_copy(ids_hbm.at[pl.ds(base, bpw)], ids_v)
        pltpu.async_copy(table_hbm.at[ids_v], rows_v, sem).wait()
        pltpu.sync_copy(rows_v, out_hbm.at[pl.ds(base, bpw)])

    return k(table, ids)


def _row_to_lane(colvec, rr, cc):
    n = colvec.shape[0]
    diag = jnp.where(rr == cc, colvec, 0.0)
    return _dotx(jnp.ones((1, n), jnp.float32), diag)


def _router_body(x_ref, w1_ref, b1_ref, w2_ref, b2_ref,
                 sid_ref, idx_ref, rls_ref, pos_ref, end0_ref,
                 ie_ref, it_ref, ilo_ref, ihi_ref):
    x = x_ref[...]
    h = jnp.maximum(_dot(x, w1_ref[...]) + b1_ref[...], 0.0)
    logits = _dot(h, w2_ref[...])[:, :E] + b2_ref[...]

    m = jnp.max(logits, axis=1, keepdims=True)
    lse = m + jnp.log(jnp.sum(jnp.exp(logits - m), axis=1, keepdims=True))
    ce = jax.lax.broadcasted_iota(jnp.int32, (B, E), 1)
    idx = jnp.min(jnp.where(logits == m, ce, E), axis=1, keepdims=True)
    idx_ref[...] = idx
    rls = m - lse

    oh = (ce == idx).astype(jnp.float32)
    counts = jnp.sum(oh, axis=0, keepdims=True)
    ree = jax.lax.broadcasted_iota(jnp.int32, (E, E), 0)
    cee = jax.lax.broadcasted_iota(jnp.int32, (E, E), 1)
    starts = _dotx(counts, (ree < cee).astype(jnp.float32))
    ends = starts + counts
    end0_ref[...] = counts[0:1, 0:1].astype(jnp.int32)

    rr = jax.lax.broadcasted_iota(jnp.int32, (B, B), 0)
    cc = jax.lax.broadcasted_iota(jnp.int32, (B, B), 1)
    idxf = idx.astype(jnp.float32)
    idx_row = _row_to_lane(idxf, rr, cc)
    eq = (idx_row == idxf) & (cc < rr)
    rank = jnp.sum(eq.astype(jnp.float32), axis=1, keepdims=True)
    start_i = jnp.sum(starts * oh, axis=1, keepdims=True)
    posf = start_i + rank
    pos_ref[...] = posf.astype(jnp.int32)

    pos_row = _row_to_lane(posf, rr, cc)
    perm = (pos_row.astype(jnp.int32) == rr).astype(jnp.float32)
    sid_ref[...] = jnp.round(
        jnp.sum(perm * cc.astype(jnp.float32), axis=1, keepdims=True)).astype(jnp.int32)
    rls_row = _row_to_lane(rls, rr, cc)
    rls_ref[...] = jnp.sum(perm * rls_row, axis=1, keepdims=True)

    nf = E * NT
    fc = jax.lax.broadcasted_iota(jnp.int32, (nf, 1), 0)
    ec = fc // NT
    tc = fc - ec * NT
    ohe = (jax.lax.broadcasted_iota(jnp.int32, (nf, E), 1) == ec).astype(jnp.float32)
    st_c = jnp.sum(ohe * starts, axis=1, keepdims=True)
    en_c = jnp.sum(ohe * ends, axis=1, keepdims=True)
    lo_c = jnp.maximum(st_c, (tc * TB).astype(jnp.float32))
    hi_c = jnp.minimum(en_c, ((tc + 1) * TB).astype(jnp.float32))
    act_c = (lo_c < hi_c).astype(jnp.float32)

    rf = jax.lax.broadcasted_iota(jnp.int32, (nf, nf), 0)
    cf = jax.lax.broadcasted_iota(jnp.int32, (nf, nf), 1)
    act_row = _row_to_lane(act_c, rf, cf)
    cix_row = _dotx(act_row, (rf < cf).astype(jnp.float32))
    total = jnp.sum(act_c)

    selt = ((cix_row == rf.astype(jnp.float32)) & (act_row > 0.5)).astype(jnp.float32)
    vals = jnp.concatenate([ec.astype(jnp.float32), tc.astype(jnp.float32),
                            lo_c, hi_c], axis=1)
    items = _dotx(selt, vals)
    sc = jax.lax.broadcasted_iota(jnp.int32, (nf, 1), 0).astype(jnp.float32)
    last = jnp.sum(jnp.where(sc == total - 1.0, items, 0.0), axis=0, keepdims=True)
    pad = sc >= total
    ie_ref[...] = jnp.round(jnp.where(pad, last[0, 0], items[:, 0:1])).astype(jnp.int32)
    it_ref[...] = jnp.round(jnp.where(pad, last[0, 1], items[:, 1:2])).astype(jnp.int32)
    ilo_ref[...] = jnp.round(jnp.where(pad, 0.0, items[:, 2:3])).astype(jnp.int32)
    ihi_ref[...] = jnp.round(jnp.where(pad, 0.0, items[:, 3:4])).astype(jnp.int32)


def _expert_finish_body(ie_ref, it_ref, ilo_ref, ihi_ref, e0_ref,
                        sx_ref, w1_ref, b1_ref, w2_ref, b2_ref,
                        rls_ref, pos_ref, out_ref, fs_ref):
    i = pl.program_id(0)
    t = it_ref[i]
    lo = ilo_ref[i]
    hi = ihi_ref[i]

    @pl.when(lo < hi)
    def _compute():
        x = sx_ref[pl.ds(t * TB, TB), :]
        h = jnp.maximum(_dot(x, w1_ref[0]) + b1_ref[0], 0.0)
        le = _dot(h, w2_ref[0]) + b2_ref[0]
        g = t * TB + jax.lax.broadcasted_iota(jnp.int32, (TB, 1), 0)
        mask = (g >= lo) & (g < hi)
        cur = fs_ref[pl.ds(t * TB, TB), :]
        fs_ref[pl.ds(t * TB, TB), :] = jnp.where(mask, le, cur)

    @pl.when(i == NITEMS - 1)
    def _finish():
        l0 = fs_ref[...]
        m0 = jnp.max(l0, axis=1, keepdims=True)
        lse = m0 + jnp.log(jnp.sum(jnp.exp(l0 - m0), axis=1, keepdims=True))
        cols = jax.lax.broadcasted_iota(jnp.int32, (B, L), 1)

        lcur = l0
        vals, cands = [], []
        for _ in range(K + 1):
            mv = jnp.max(lcur, axis=1, keepdims=True)
            am = jnp.min(jnp.where(lcur == mv, cols, L), axis=1, keepdims=True)
            vals.append(mv)
            cands.append(am)
            lcur = jnp.where(cols == am, NEG, lcur)
        v11 = jnp.concatenate(vals, axis=1)
        c11 = jnp.concatenate(cands, axis=1)

        rowi = jax.lax.broadcasted_iota(jnp.int32, (B, 1), 0)
        inval = (rowi < e0_ref[0]) & (c11 == 0)
        j11 = jax.lax.broadcasted_iota(jnp.int32, (B, K + 1), 1)
        pinv = jnp.min(jnp.where(inval, j11, K + 1), axis=1, keepdims=True)
        j10 = jax.lax.broadcasted_iota(jnp.int32, (B, K), 1)
        shift = j10 >= pinv
        kept = jnp.where(shift, c11[:, 1:K + 1], c11[:, :K])
        keptv = jnp.where(shift, v11[:, 1:K + 1], v11[:, :K])
        olp = rls_ref[...] + (keptv - lse)

        ccb = jax.lax.broadcasted_iota(jnp.int32, (B, B), 1)
        u = (ccb == pos_ref[...]).astype(jnp.float32)
        payload = jnp.concatenate(
            [olp, kept.astype(jnp.float32),
             jnp.zeros((B, 12), jnp.float32)], axis=1)
        out_ref[...] = _dotx(u, payload)


@jax.jit
def kernel(state, root_W1, root_b1, root_W2, root_b2,
           exp_W1, exp_b1, exp_W2, exp_b2):
    f32 = jnp.float32
    i32 = jnp.int32
    nf = E * NT

    router = pl.pallas_call(
        _router_body,
        out_shape=(
            jax.ShapeDtypeStruct((B, 1), i32),
            jax.ShapeDtypeStruct((B, 1), i32),
            jax.ShapeDtypeStruct((B, 1), f32),
            jax.ShapeDtypeStruct((B, 1), i32),
            jax.ShapeDtypeStruct((1, 1), i32),
            jax.ShapeDtypeStruct((nf, 1), i32),
            jax.ShapeDtypeStruct((nf, 1), i32),
            jax.ShapeDtypeStruct((nf, 1), i32),
            jax.ShapeDtypeStruct((nf, 1), i32),
        ),
    )
    root_W2p = jnp.concatenate(
        [root_W2, jnp.zeros((H, 128 - E), f32)], axis=1)
    sid, idx, rls, pos, end0, ie, it, ilo, ihi = router(
        state, root_W1, root_b1.reshape(1, H), root_W2p, root_b2.reshape(1, E))
    sx = _sc_gather(state, sid.reshape(B))

    fused = pl.pallas_call(
        _expert_finish_body,
        grid_spec=pltpu.PrefetchScalarGridSpec(
            num_scalar_prefetch=5,
            grid=(NITEMS,),
            in_specs=[
                pl.BlockSpec((B, D), lambda i, *_: (0, 0)),
                pl.BlockSpec((1, D, H), lambda i, ie, it, lo, hi, e0: (ie[i], 0, 0)),
                pl.BlockSpec((1, 1, H), lambda i, ie, it, lo, hi, e0: (ie[i], 0, 0)),
                pl.BlockSpec((1, H, L), lambda i, ie, it, lo, hi, e0: (ie[i], 0, 0)),
                pl.BlockSpec((1, 1, L), lambda i, ie, it, lo, hi, e0: (ie[i], 0, 0)),
                pl.BlockSpec((B, 1), lambda i, *_: (0, 0)),
                pl.BlockSpec((B, 1), lambda i, *_: (0, 0)),
            ],
            out_specs=pl.BlockSpec((B, 32), lambda i, *_: (0, 0)),
            scratch_shapes=[pltpu.VMEM((B, L), f32)],
        ),
        out_shape=jax.ShapeDtypeStruct((B, 32), f32),
    )
    payload = fused(ie[:NITEMS, 0], it[:NITEMS, 0], ilo[:NITEMS, 0],
                    ihi[:NITEMS, 0], end0.reshape(1), sx,
                    exp_W1, exp_b1.reshape(E, 1, H),
                    exp_W2, exp_b2.reshape(E, 1, L),
                    rls, pos)

    out_lp = payload[:, :K]
    leaf = jnp.round(payload[:, K:2 * K]).astype(i32)
    branch = jnp.broadcast_to(idx, (B, K))
    trajectories = jnp.stack([branch, leaf], axis=-1)
    return trajectories, out_lp


# speedup vs baseline: 3.0141x; 1.0012x over previous
import functools

import jax
import jax.numpy as jnp
from jax.experimental import pallas as pl
from jax.experimental.pallas import tpu as pltpu
from jax.experimental.pallas import tpu_sc as plsc

E = 16
L = 1024
D = 1024
H = 512
K = 10
B = 1024

TB = 256
NT = B // TB
NITEMS = NT + E - 1
NEG = -jnp.inf

_SC_WORKERS = 32


def _dot(a, b):
    return jnp.dot(a, b, preferred_element_type=jnp.float32)


def _dotx(a, b):
    return jnp.dot(a, b, preferred_element_type=jnp.float32,
                   precision=jax.lax.Precision.HIGHEST)


def _sc_gather(table, ids):
    bn, dn = table.shape
    bpw = bn // _SC_WORKERS
    mesh = plsc.VectorSubcoreMesh(core_axis_name="c", subcore_axis_name="s")

    @functools.partial(
        pl.kernel, mesh=mesh,
        out_type=jax.ShapeDtypeStruct((bn, dn), table.dtype),
        scratch_types=[
            pltpu.VMEM((bpw,), jnp.int32),
            pltpu.VMEM((bpw, dn), table.dtype),
            pltpu.SemaphoreType.DMA,
        ],
    )
    def k(table_hbm, ids_hbm, out_hbm, ids_v, rows_v, sem):
        wid = jax.lax.axis_index("s") * 2 + jax.lax.axis_index("c")
        base = wid * bpw
        pltpu.sync_copy(ids_hbm.at[pl.ds(base, bpw)], ids_v)
        pltpu.async_copy(table_hbm.at[ids_v], rows_v, sem).wait()
        pltpu.sync_copy(rows_v, out_hbm.at[pl.ds(base, bpw)])

    return k(table, ids)


def _row_to_lane(colvec, rr, cc):
    n = colvec.shape[0]
    diag = jnp.where(rr == cc, colvec, 0.0)
    return _dotx(jnp.ones((1, n), jnp.float32), diag)


def _router_body(x_ref, w1_ref, b1_ref, w2_ref, b2_ref,
                 sid_ref, idx_ref, rls_ref, pos_ref, end0_ref,
                 ie_ref, it_ref, ilo_ref, ihi_ref):
    x = x_ref[...]
    h = jnp.maximum(_dot(x, w1_ref[...]) + b1_ref[...], 0.0)
    logits = _dot(h, w2_ref[...])[:, :E] + b2_ref[...]

    m = jnp.max(logits, axis=1, keepdims=True)
    lse = m + jnp.log(jnp.sum(jnp.exp(logits - m), axis=1, keepdims=True))
    ce = jax.lax.broadcasted_iota(jnp.int32, (B, E), 1)
    idx = jnp.min(jnp.where(logits == m, ce, E), axis=1, keepdims=True)
    idx_ref[...] = idx
    rls = m - lse

    oh = (ce == idx).astype(jnp.float32)
    counts = jnp.sum(oh, axis=0, keepdims=True)
    ree = jax.lax.broadcasted_iota(jnp.int32, (E, E), 0)
    cee = jax.lax.broadcasted_iota(jnp.int32, (E, E), 1)
    starts = _dotx(counts, (ree < cee).astype(jnp.float32))
    ends = starts + counts
    end0_ref[...] = counts[0:1, 0:1].astype(jnp.int32)

    rr = jax.lax.broadcasted_iota(jnp.int32, (B, B), 0)
    cc = jax.lax.broadcasted_iota(jnp.int32, (B, B), 1)
    idxf = idx.astype(jnp.float32)
    idx_row = _row_to_lane(idxf, rr, cc)
    eq = (idx_row == idxf) & (cc < rr)
    rank = jnp.sum(eq.astype(jnp.float32), axis=1, keepdims=True)
    start_i = jnp.sum(starts * oh, axis=1, keepdims=True)
    posf = start_i + rank
    pos_ref[...] = posf.astype(jnp.int32)

    pos_row = _row_to_lane(posf, rr, cc)
    hit = pos_row.astype(jnp.int32) == rr
    sid_ref[...] = jnp.sum(jnp.where(hit, cc, 0), axis=1, keepdims=True)
    rls_row = _row_to_lane(rls, rr, cc)
    rls_ref[...] = jnp.sum(jnp.where(hit, rls_row, 0.0), axis=1, keepdims=True)

    nf = E * NT
    fc = jax.lax.broadcasted_iota(jnp.int32, (nf, 1), 0)
    ec = fc // NT
    tc = fc - ec * NT
    ohe = (jax.lax.broadcasted_iota(jnp.int32, (nf, E), 1) == ec).astype(jnp.float32)
    st_c = jnp.sum(ohe * starts, axis=1, keepdims=True)
    en_c = jnp.sum(ohe * ends, axis=1, keepdims=True)
    lo_c = jnp.maximum(st_c, (tc * TB).astype(jnp.float32))
    hi_c = jnp.minimum(en_c, ((tc + 1) * TB).astype(jnp.float32))
    act_c = (lo_c < hi_c).astype(jnp.float32)

    rf = jax.lax.broadcasted_iota(jnp.int32, (nf, nf), 0)
    cf = jax.lax.broadcasted_iota(jnp.int32, (nf, nf), 1)
    act_row = _row_to_lane(act_c, rf, cf)
    cix_row = _dotx(act_row, (rf < cf).astype(jnp.float32))
    total = jnp.sum(act_c)

    selt = ((cix_row == rf.astype(jnp.float32)) & (act_row > 0.5)).astype(jnp.float32)
    vals = jnp.concatenate([ec.astype(jnp.float32), tc.astype(jnp.float32),
                            lo_c, hi_c], axis=1)
    items = _dotx(selt, vals)
    sc = jax.lax.broadcasted_iota(jnp.int32, (nf, 1), 0).astype(jnp.float32)
    last = jnp.sum(jnp.where(sc == total - 1.0, items, 0.0), axis=0, keepdims=True)
    pad = sc >= total
    ie_ref[...] = jnp.round(jnp.where(pad, last[0, 0], items[:, 0:1])).astype(jnp.int32)
    it_ref[...] = jnp.round(jnp.where(pad, last[0, 1], items[:, 1:2])).astype(jnp.int32)
    ilo_ref[...] = jnp.round(jnp.where(pad, 0.0, items[:, 2:3])).astype(jnp.int32)
    ihi_ref[...] = jnp.round(jnp.where(pad, 0.0, items[:, 3:4])).astype(jnp.int32)


def _expert_finish_body(ie_ref, it_ref, ilo_ref, ihi_ref, e0_ref,
                        sx_ref, w1_ref, b1_ref, w2_ref, b2_ref,
                        rls_ref, pos_ref, out_ref, fs_ref):
    i = pl.program_id(0)
    t = it_ref[i]
    lo = ilo_ref[i]
    hi = ihi_ref[i]

    @pl.when(lo < hi)
    def _compute():
        x = sx_ref[pl.ds(t * TB, TB), :]
        h = jnp.maximum(_dot(x, w1_ref[0]) + b1_ref[0], 0.0)
        le = _dot(h, w2_ref[0]) + b2_ref[0]
        g = t * TB + jax.lax.broadcasted_iota(jnp.int32, (TB, 1), 0)
        mask = (g >= lo) & (g < hi)
        cur = fs_ref[pl.ds(t * TB, TB), :]
        fs_ref[pl.ds(t * TB, TB), :] = jnp.where(mask, le, cur)

    @pl.when(i == NITEMS - 1)
    def _finish():
        l0 = fs_ref[...]
        m0 = jnp.max(l0, axis=1, keepdims=True)
        lse = m0 + jnp.log(jnp.sum(jnp.exp(l0 - m0), axis=1, keepdims=True))
        cols = jax.lax.broadcasted_iota(jnp.int32, (B, L), 1)

        lcur = l0
        vals, cands = [], []
        for _ in range(K + 1):
            mv = jnp.max(lcur, axis=1, keepdims=True)
            am = jnp.min(jnp.where(lcur == mv, cols, L), axis=1, keepdims=True)
            vals.append(mv)
            cands.append(am)
            lcur = jnp.where(cols == am, NEG, lcur)
        v11 = jnp.concatenate(vals, axis=1)
        c11 = jnp.concatenate(cands, axis=1)

        rowi = jax.lax.broadcasted_iota(jnp.int32, (B, 1), 0)
        inval = (rowi < e0_ref[0]) & (c11 == 0)
        j11 = jax.lax.broadcasted_iota(jnp.int32, (B, K + 1), 1)
        pinv = jnp.min(jnp.where(inval, j11, K + 1), axis=1, keepdims=True)
        j10 = jax.lax.broadcasted_iota(jnp.int32, (B, K), 1)
        shift = j10 >= pinv
        kept = jnp.where(shift, c11[:, 1:K + 1], c11[:, :K])
        keptv = jnp.where(shift, v11[:, 1:K + 1], v11[:, :K])
        olp = rls_ref[...] + (keptv - lse)

        ccb = jax.lax.broadcasted_iota(jnp.int32, (B, B), 1)
        u = (ccb == pos_ref[...]).astype(jnp.float32)
        payload = jnp.concatenate(
            [olp, kept.astype(jnp.float32),
             jnp.zeros((B, 12), jnp.float32)], axis=1)
        out_ref[...] = _dotx(u, payload)


@jax.jit
def kernel(state, root_W1, root_b1, root_W2, root_b2,
           exp_W1, exp_b1, exp_W2, exp_b2):
    f32 = jnp.float32
    i32 = jnp.int32
    nf = E * NT

    router = pl.pallas_call(
        _router_body,
        out_shape=(
            jax.ShapeDtypeStruct((B, 1), i32),
            jax.ShapeDtypeStruct((B, 1), i32),
            jax.ShapeDtypeStruct((B, 1), f32),
            jax.ShapeDtypeStruct((B, 1), i32),
            jax.ShapeDtypeStruct((1, 1), i32),
            jax.ShapeDtypeStruct((nf, 1), i32),
            jax.ShapeDtypeStruct((nf, 1), i32),
            jax.ShapeDtypeStruct((nf, 1), i32),
            jax.ShapeDtypeStruct((nf, 1), i32),
        ),
    )
    root_W2p = jnp.concatenate(
        [root_W2, jnp.zeros((H, 128 - E), f32)], axis=1)
    sid, idx, rls, pos, end0, ie, it, ilo, ihi = router(
        state, root_W1, root_b1.reshape(1, H), root_W2p, root_b2.reshape(1, E))
    sx = _sc_gather(state, sid.reshape(B))

    fused = pl.pallas_call(
        _expert_finish_body,
        grid_spec=pltpu.PrefetchScalarGridSpec(
            num_scalar_prefetch=5,
            grid=(NITEMS,),
            in_specs=[
                pl.BlockSpec((B, D), lambda i, *_: (0, 0)),
                pl.BlockSpec((1, D, H), lambda i, ie, it, lo, hi, e0: (ie[i], 0, 0)),
                pl.BlockSpec((1, 1, H), lambda i, ie, it, lo, hi, e0: (ie[i], 0, 0)),
                pl.BlockSpec((1, H, L), lambda i, ie, it, lo, hi, e0: (ie[i], 0, 0)),
                pl.BlockSpec((1, 1, L), lambda i, ie, it, lo, hi, e0: (ie[i], 0, 0)),
                pl.BlockSpec((B, 1), lambda i, *_: (0, 0)),
                pl.BlockSpec((B, 1), lambda i, *_: (0, 0)),
            ],
            out_specs=pl.BlockSpec((B, 32), lambda i, *_: (0, 0)),
            scratch_shapes=[pltpu.VMEM((B, L), f32)],
        ),
        out_shape=jax.ShapeDtypeStruct((B, 32), f32),
    )
    payload = fused(ie[:NITEMS, 0], it[:NITEMS, 0], ilo[:NITEMS, 0],
                    ihi[:NITEMS, 0], end0.reshape(1), sx,
                    exp_W1, exp_b1.reshape(E, 1, H),
                    exp_W2, exp_b2.reshape(E, 1, L),
                    rls, pos)

    out_lp = payload[:, :K]
    leaf = jnp.round(payload[:, K:2 * K]).astype(i32)
    branch = jnp.broadcast_to(idx, (B, K))
    trajectories = jnp.stack([branch, leaf], axis=-1)
    return trajectories, out_lp
